# baseline (speedup 1.0000x reference)
import jax
import jax.numpy as jnp
from jax import lax
from jax.experimental import pallas as pl
from jax.experimental.pallas import tpu as pltpu
from jax.experimental.pallas import tpu_sc as plsc

_G = 1024
_E = 6400000
_NEG = -1e30
_NW = 32
_PER_W = _E // _NW
_CHUNK = 10000
_ROUNDS = _PER_W // _CHUNK
_ITERS = _CHUNK // 16
_TROWS = _G + 1


def _sc_body(xs_hbm, sb_hbm, vs_hbm, wmax_hbm, wsum_hbm,
             xbuf, sbuf, vbuf, accm, accs, semx, sems, semv):
    wid = lax.axis_index("c") * 16 + lax.axis_index("s")
    base = wid * _PER_W
    lanes = lax.iota(jnp.int32, 16)
    lanes16 = lanes + 16
    negv = jnp.full((16,), _NEG, jnp.float32)
    zerov = jnp.zeros((16,), jnp.float32)

    def init_row(i, c):
        accm[pl.ds(i * 16, 16)] = negv
        accs[pl.ds(i * 16, 16)] = zerov
        return c

    lax.fori_loop(0, _TROWS, init_row, 0)

    def start_round(r):
        sel = lax.rem(r, 2) * _CHUNK
        off = base + r * _CHUNK
        pltpu.async_copy(xs_hbm.at[pl.ds(off, _CHUNK)],
                         xbuf.at[pl.ds(sel, _CHUNK)], semx)
        pltpu.async_copy(sb_hbm.at[pl.ds(off, _CHUNK)],
                         sbuf.at[pl.ds(sel, _CHUNK)], sems)
        pltpu.async_copy(vs_hbm.at[pl.ds(off, _CHUNK)],
                         vbuf.at[pl.ds(sel, _CHUNK)], semv)

    def wait_round(r):
        sel = lax.rem(r, 2) * _CHUNK
        off = base + r * _CHUNK
        pltpu.make_async_copy(xs_hbm.at[pl.ds(off, _CHUNK)],
                              xbuf.at[pl.ds(sel, _CHUNK)], semx).wait()
        pltpu.make_async_copy(sb_hbm.at[pl.ds(off, _CHUNK)],
                              sbuf.at[pl.ds(sel, _CHUNK)], sems).wait()
        pltpu.make_async_copy(vs_hbm.at[pl.ds(off, _CHUNK)],
                              vbuf.at[pl.ds(sel, _CHUNK)], semv).wait()

    start_round(0)

    def round_body(r, carry):
        wait_round(r)

        @pl.when(r + 1 < _ROUNDS)
        def _():
            start_round(r + 1)

        sel = lax.rem(r, 2) * _CHUNK

        def it(i, ci):
            prev_sg, runm, runs, prev_id = ci
            o0 = sel + i * 80
            tailv = sbuf[pl.ds(o0 + 64, 16)]
            last_id = tailv[15]

            def common(args):
                prev_sg, runm, runs = args
                for u in range(5):
                    o = o0 + u * 16
                    x = xbuf[pl.ds(o, 16)]
                    v = vbuf[pl.ds(o, 16)]
                    ok = v > 0.0
                    e = jnp.where(ok, jnp.exp(x), 0.0)
                    xm = jnp.where(ok, x, _NEG)
                    runm = jnp.maximum(runm, xm)
                    runs = runs + e
                return prev_sg, runm, runs

            def boundary(args):
                prev_sg, runm, runs = args
                for u in range(5):
                    o = o0 + u * 16
                    x = xbuf[pl.ds(o, 16)]
                    sg = sbuf[pl.ds(o, 16)]
                    v = vbuf[pl.ds(o, 16)]
                    ok = v > 0.0
                    e = jnp.where(ok, jnp.exp(x), 0.0)
                    xm = jnp.where(ok, x, _NEG)
                    flush = sg != prev_sg
                    idx = prev_sg * 16 + lanes16
                    plsc.store_scatter(accm, [idx], runm, mask=flush)
                    plsc.store_scatter(accs, [idx], runs, mask=flush)
                    runm = jnp.where(flush, xm, jnp.maximum(runm, xm))
                    runs = jnp.where(flush, e, runs + e)
                    prev_sg = sg
                return prev_sg, runm, runs

            same = last_id == prev_id
            prev_sg, runm, runs = lax.cond(
                same, common, boundary, (prev_sg, runm, runs))
            tail_uniform = tailv[0] == last_id
            prev_id = jnp.where(
                same, prev_id, jnp.where(tail_uniform, last_id, -2))
            return prev_sg, runm, runs, prev_id

        return lax.fori_loop(0, _ITERS // 5, it, carry)

    prev_sg = jnp.full((16,), -1, jnp.int32)
    prev_sg, runm, runs, _ = lax.fori_loop(
        0, _ROUNDS, round_body, (prev_sg, negv, zerov, jnp.int32(-1)))

    idx = prev_sg * 16 + lanes16
    plsc.store_scatter(accm, [idx], runm)
    plsc.store_scatter(accs, [idx], runs)

    pltpu.sync_copy(accm, wmax_hbm.at[wid])
    pltpu.sync_copy(accs, wsum_hbm.at[wid])


def _combine_body(wmax_ref, wsum_ref, sx_ref, w_ref, b_ref, out_ref):
    wmax = wmax_ref[...][:, 1:, :]
    wsum = wsum_ref[...][:, 1:, :]
    seg_max = jnp.max(jnp.max(wmax, axis=0), axis=1)
    seg_sum = jnp.sum(jnp.sum(wsum, axis=0), axis=1)
    stop = jnp.sum(sx_ref[...] * w_ref[...][:, 0][None, :], axis=1) + b_ref[0]
    edge_lse = jnp.where(seg_sum > 0.0, jnp.log(seg_sum), _NEG)
    m = jnp.maximum(seg_max, stop)
    m2 = jnp.maximum(edge_lse, stop)
    lse = m2 + jnp.log(jnp.exp(edge_lse - m2) + jnp.exp(stop - m2))
    out_ref[...] = m - lse


def kernel(edge_scores, state_x, W_stop, b_stop, edge_batch, valid_edges):
    v32 = valid_edges.astype(jnp.float32)

    mesh = plsc.VectorSubcoreMesh(core_axis_name="c", subcore_axis_name="s",
                                  num_cores=2, num_subcores=16)
    sc = pl.kernel(
        _sc_body,
        out_type=(
            jax.ShapeDtypeStruct((_NW, _TROWS * 16), jnp.float32),
            jax.ShapeDtypeStruct((_NW, _TROWS * 16), jnp.float32),
        ),
        mesh=mesh,
        compiler_params=pltpu.CompilerParams(needs_layout_passes=False),
        scratch_types=[
            pltpu.VMEM((2 * _CHUNK,), jnp.float32),
            pltpu.VMEM((2 * _CHUNK,), jnp.int32),
            pltpu.VMEM((2 * _CHUNK,), jnp.float32),
            pltpu.VMEM((_TROWS * 16,), jnp.float32),
            pltpu.VMEM((_TROWS * 16,), jnp.float32),
            pltpu.SemaphoreType.DMA,
            pltpu.SemaphoreType.DMA,
            pltpu.SemaphoreType.DMA,
        ],
    )
    wmax, wsum = sc(edge_scores, edge_batch, v32)
    wmax = wmax.reshape(_NW, _TROWS, 16)
    wsum = wsum.reshape(_NW, _TROWS, 16)

    out = pl.pallas_call(
        _combine_body,
        out_shape=jax.ShapeDtypeStruct((_G,), jnp.float32),
    )(wmax, wsum, state_x, W_stop, b_stop)
    return out


# speedup vs baseline: 1.0945x; 1.0945x over previous
import jax
import jax.numpy as jnp
from jax import lax
from jax.experimental import pallas as pl
from jax.experimental.pallas import tpu as pltpu
from jax.experimental.pallas import tpu_sc as plsc

_G = 1024
_E = 6400000
_NEG = -1e30
_NW = 32
_PER_W = _E // _NW
_CHUNK = 8000
_ROUNDS = _PER_W // _CHUNK
_ITERS = _CHUNK // 16
_TROWS = _G + 1


def _sc_body(xs_hbm, sb_hbm, vs_hbm, wmax_hbm, wsum_hbm,
             xbuf, sbuf, vbuf, accm, accs, semx, sems, semv):
    wid = lax.axis_index("c") * 16 + lax.axis_index("s")
    base = wid * _PER_W
    lanes = lax.iota(jnp.int32, 16)
    lanes16 = lanes + 16
    negv = jnp.full((16,), _NEG, jnp.float32)
    zerov = jnp.zeros((16,), jnp.float32)

    def init_row(i, c):
        accm[pl.ds(i * 16, 16)] = negv
        accs[pl.ds(i * 16, 16)] = zerov
        return c

    lax.fori_loop(0, _TROWS, init_row, 0)

    def start_round(r):
        sel = lax.rem(r, 2) * _CHUNK
        off = base + r * _CHUNK
        pltpu.async_copy(xs_hbm.at[pl.ds(off, _CHUNK)],
                         xbuf.at[pl.ds(sel, _CHUNK)], semx)
        pltpu.async_copy(sb_hbm.at[pl.ds(off, _CHUNK)],
                         sbuf.at[pl.ds(sel, _CHUNK)], sems)
        pltpu.async_copy(vs_hbm.at[pl.ds(off, _CHUNK)],
                         vbuf.at[pl.ds(sel, _CHUNK)], semv)

    def wait_round(r):
        sel = lax.rem(r, 2) * _CHUNK
        off = base + r * _CHUNK
        pltpu.make_async_copy(xs_hbm.at[pl.ds(off, _CHUNK)],
                              xbuf.at[pl.ds(sel, _CHUNK)], semx).wait()
        pltpu.make_async_copy(sb_hbm.at[pl.ds(off, _CHUNK)],
                              sbuf.at[pl.ds(sel, _CHUNK)], sems).wait()
        pltpu.make_async_copy(vs_hbm.at[pl.ds(off, _CHUNK)],
                              vbuf.at[pl.ds(sel, _CHUNK)], semv).wait()

    start_round(0)

    def round_body(r, carry):
        wait_round(r)

        @pl.when(r + 1 < _ROUNDS)
        def _():
            start_round(r + 1)

        sel = lax.rem(r, 2) * _CHUNK

        def it(i, ci):
            prev_sg, runm, runs = ci
            for u in range(10):
                o = sel + i * 160 + u * 16
                x = xbuf[pl.ds(o, 16)]
                sg = sbuf[pl.ds(o, 16)]
                v = vbuf[pl.ds(o, 16)]
                ok = v > 0.0
                e = jnp.where(ok, jnp.exp(x), 0.0)
                xm = jnp.where(ok, x, _NEG)
                flush = sg != prev_sg
                idx = prev_sg * 16 + lanes16
                plsc.store_scatter(accm, [idx], runm, mask=flush)
                plsc.store_scatter(accs, [idx], runs, mask=flush)
                runm = jnp.where(flush, xm, jnp.maximum(runm, xm))
                runs = jnp.where(flush, e, runs + e)
                prev_sg = sg
            return prev_sg, runm, runs

        return lax.fori_loop(0, _ITERS // 10, it, carry)

    prev_sg = jnp.full((16,), -1, jnp.int32)
    prev_sg, runm, runs = lax.fori_loop(
        0, _ROUNDS, round_body, (prev_sg, negv, zerov))

    idx = prev_sg * 16 + lanes16
    plsc.store_scatter(accm, [idx], runm)
    plsc.store_scatter(accs, [idx], runs)

    pltpu.sync_copy(accm, wmax_hbm.at[wid])
    pltpu.sync_copy(accs, wsum_hbm.at[wid])


def _combine_body(wmax_ref, wsum_ref, sx_ref, w_ref, b_ref, out_ref):
    wmax = wmax_ref[...][:, 1:, :]
    wsum = wsum_ref[...][:, 1:, :]
    seg_max = jnp.max(jnp.max(wmax, axis=0), axis=1)
    seg_sum = jnp.sum(jnp.sum(wsum, axis=0), axis=1)
    stop = jnp.sum(sx_ref[...] * w_ref[...][:, 0][None, :], axis=1) + b_ref[0]
    edge_lse = jnp.where(seg_sum > 0.0, jnp.log(seg_sum), _NEG)
    m = jnp.maximum(seg_max, stop)
    m2 = jnp.maximum(edge_lse, stop)
    lse = m2 + jnp.log(jnp.exp(edge_lse - m2) + jnp.exp(stop - m2))
    out_ref[...] = m - lse


def kernel(edge_scores, state_x, W_stop, b_stop, edge_batch, valid_edges):
    v32 = valid_edges.astype(jnp.float32)

    mesh = plsc.VectorSubcoreMesh(core_axis_name="c", subcore_axis_name="s",
                                  num_cores=2, num_subcores=16)
    sc = pl.kernel(
        _sc_body,
        out_type=(
            jax.ShapeDtypeStruct((_NW, _TROWS * 16), jnp.float32),
            jax.ShapeDtypeStruct((_NW, _TROWS * 16), jnp.float32),
        ),
        mesh=mesh,
        compiler_params=pltpu.CompilerParams(needs_layout_passes=False),
        scratch_types=[
            pltpu.VMEM((2 * _CHUNK,), jnp.float32),
            pltpu.VMEM((2 * _CHUNK,), jnp.int32),
            pltpu.VMEM((2 * _CHUNK,), jnp.float32),
            pltpu.VMEM((_TROWS * 16,), jnp.float32),
            pltpu.VMEM((_TROWS * 16,), jnp.float32),
            pltpu.SemaphoreType.DMA,
            pltpu.SemaphoreType.DMA,
            pltpu.SemaphoreType.DMA,
        ],
    )
    wmax, wsum = sc(edge_scores, edge_batch, v32)
    wmax = wmax.reshape(_NW, _TROWS, 16)
    wsum = wsum.reshape(_NW, _TROWS, 16)

    out = pl.pallas_call(
        _combine_body,
        out_shape=jax.ShapeDtypeStruct((_G,), jnp.float32),
    )(wmax, wsum, state_x, W_stop, b_stop)
    return out
